# Initial kernel scaffold; baseline (speedup 1.0000x reference)
import jax
import jax.numpy as jnp
from jax.experimental import pallas as pl


def kernel(x, edge_index, W1, b1, W2, b2, gamma, beta, Wl, bl):
    raise NotImplementedError("write your pallas kernel here")


# speedup vs baseline: 7.9853x; 7.9853x over previous
import functools

import jax
import jax.numpy as jnp
from jax import lax
from jax.experimental import pallas as pl
from jax.experimental.pallas import tpu as pltpu
from jax.experimental.pallas import tpu_sc as plsc

N = 10000
D = 128
H = 128
NP = 10240
NC = 2
NS = 16
L = 16
NW = NC * NS
CHUNK = 128


def _sc_degrees(comb):
    C = comb.shape[1]
    mesh = plsc.VectorSubcoreMesh(core_axis_name="c", subcore_axis_name="s")

    @functools.partial(
        pl.kernel,
        out_type=(jax.ShapeDtypeStruct((NW, NP), jnp.float32),
                  jax.ShapeDtypeStruct((NW, NP), jnp.float32)),
        mesh=mesh,
        compiler_params=pltpu.CompilerParams(needs_layout_passes=False),
        scratch_types=[
            pltpu.VMEM((C, 2, CHUNK), jnp.int32),
            pltpu.VMEM((NP,), jnp.float32),
            pltpu.VMEM((NP,), jnp.float32),
        ],
    )
    def k(comb_hbm, outO, outI, idx_v, dO_v, dI_v):
        cid = lax.axis_index("c")
        sid = lax.axis_index("s")
        wid = sid * NC + cid
        pltpu.sync_copy(comb_hbm.at[wid], idx_v)

        zeros = jnp.zeros((L,), jnp.float32)

        def zbody(i, _):
            dO_v[pl.ds(i * L, L)] = zeros
            dI_v[pl.ds(i * L, L)] = zeros
            return 0

        lax.fori_loop(0, NP // L, zbody, 0)

        ones = jnp.ones((L,), jnp.float32)

        def body(c, _):
            for g in range(CHUNK // L):
                s16 = idx_v[c, 0, pl.ds(g * L, L)]
                plsc.addupdate_scatter(dO_v, [s16], ones)
                d16 = idx_v[c, 1, pl.ds(g * L, L)]
                plsc.addupdate_scatter(dI_v, [d16], ones)
            return 0

        lax.fori_loop(0, C, body, 0)
        pltpu.sync_copy(dO_v, outO.at[wid])
        pltpu.sync_copy(dI_v, outI.at[wid])

    return k(comb)


def _tc_prep(x, W1, gamma, beta, degOp, degIp):
    def body(x_ref, w1_ref, g_ref, b_ref, dO_ref, dI_ref, u_ref, st_ref):
        xv = x_ref[...]
        mean = jnp.mean(xv, axis=0, keepdims=True)
        var = jnp.mean((xv - mean) ** 2, axis=0, keepdims=True)
        h = (xv - mean) * lax.rsqrt(var + 1e-5) * g_ref[...][None, :] \
            + b_ref[...][None, :]
        y = jnp.dot(h, w1_ref[...], preferred_element_type=jnp.float32)

        degO = jnp.sum(dO_ref[...], axis=0)
        degI = jnp.sum(dI_ref[...], axis=0)
        ns = lax.rsqrt(jnp.where(degO > 0, degO, 1.0))
        nd = lax.rsqrt(jnp.where(degI > 0, degI, 1.0))
        st_ref[0, :] = ns
        st_ref[1, :] = nd

        u_ref[0:N, :] = y * ns[0:N][:, None]
        u_ref[N:NP, :] = jnp.zeros((NP - N, H), jnp.float32)

    return pl.pallas_call(
        body,
        out_shape=(jax.ShapeDtypeStruct((NP, H), jnp.float32),
                   jax.ShapeDtypeStruct((2, NP), jnp.float32)),
    )(x, W1, gamma, beta, degOp, degIp)


def _sc_aggregate(comb, u, nd):
    C = comb.shape[1]
    assert C % 2 == 0
    R = NP // NS
    mesh = plsc.VectorSubcoreMesh(core_axis_name="c", subcore_axis_name="s")

    @functools.partial(
        pl.kernel,
        out_type=(jax.ShapeDtypeStruct((NC, NP, H), jnp.float32),
                  jax.ShapeDtypeStruct((NC, NP), jnp.float32)),
        mesh=mesh,
        compiler_params=pltpu.CompilerParams(needs_layout_passes=False),
        scratch_types=[
            pltpu.VMEM((2, CHUNK), jnp.int32),
            pltpu.VMEM((2, CHUNK), jnp.int32),
            pltpu.VMEM((CHUNK,), jnp.float32),
            pltpu.VMEM((CHUNK,), jnp.float32),
            pltpu.VMEM((CHUNK, H), jnp.float32),
            pltpu.VMEM((CHUNK, H), jnp.float32),
            pltpu.VMEM_SHARED((NP, H), jnp.float32),
            pltpu.VMEM_SHARED((NP,), jnp.float32),
            pltpu.SemaphoreType.DMA,
            pltpu.SemaphoreType.DMA,
            pltpu.SemaphoreType.DMA,
            pltpu.SemaphoreType.DMA,
            pltpu.SemaphoreType.DMA,
            pltpu.SemaphoreType.DMA,
        ],
    )
    def k(comb_hbm, u_hbm, nd_hbm, z_out, w_out,
          idx0, idx1, ndv0, ndv1, bufA, bufB, zacc, wacc,
          semA, semB, semI0, semI1, semN0, semN1):
        cid = lax.axis_index("c")
        sid = lax.axis_index("s")
        wid = sid * NC + cid

        zeros = jnp.zeros((L,), jnp.float32)

        def zb(r, _):
            for g in range(H // L):
                bufA[r, pl.ds(g * L, L)] = zeros
            return 0

        lax.fori_loop(0, CHUNK, zb, 0)

        for b in range(R // CHUNK):
            pltpu.sync_copy(bufA, zacc.at[pl.ds(sid * R + b * CHUNK, CHUNK)])
            pltpu.sync_copy(bufA.at[0],
                            wacc.at[pl.ds(sid * R + b * CHUNK, CHUNK)])

        pltpu.sync_copy(comb_hbm.at[wid, 0], idx0)
        pltpu.async_copy(u_hbm.at[idx0.at[0]], bufA, semA)
        pltpu.async_copy(comb_hbm.at[wid, 1], idx1, semI1)
        plsc.subcore_barrier()

        def uwait(buf, sem):
            pltpu.make_async_copy(u_hbm.at[pl.ds(0, CHUNK)], buf, sem).wait()

        def ndwait(ndv, sem):
            pltpu.make_async_copy(nd_hbm.at[pl.ds(0, CHUNK)], ndv, sem).wait()

        def idxwait(idxb, sem):
            pltpu.make_async_copy(comb_hbm.at[0, 0], idxb, sem).wait()

        last = C // 2 - 1

        def body(i, _):
            c0 = 2 * i
            c1 = c0 + 1
            idxwait(idx1, semI1)
            pltpu.async_copy(u_hbm.at[idx1.at[0]], bufB, semB)

            pltpu.async_copy(nd_hbm.at[idx0.at[1]], ndv0, semN0)
            uwait(bufA, semA)
            pltpu.sync_copy(bufA, zacc.at[idx0.at[1]], add=True)
            ndwait(ndv0, semN0)
            pltpu.sync_copy(ndv0, wacc.at[idx0.at[0]], add=True)

            @pl.when(i < last)
            def _():
                pltpu.async_copy(comb_hbm.at[wid, c0 + 2], idx0, semI0)

            pltpu.async_copy(nd_hbm.at[idx1.at[1]], ndv1, semN1)

            @pl.when(i < last)
            def _():
                idxwait(idx0, semI0)
                pltpu.async_copy(u_hbm.at[idx0.at[0]], bufA, semA)

            uwait(bufB, semB)
            pltpu.sync_copy(bufB, zacc.at[idx1.at[1]], add=True)
            ndwait(ndv1, semN1)
            pltpu.sync_copy(ndv1, wacc.at[idx1.at[0]], add=True)

            @pl.when(i < last)
            def _():
                pltpu.async_copy(comb_hbm.at[wid, c1 + 2], idx1, semI1)

            return 0

        lax.fori_loop(0, C // 2, body, 0)

        plsc.subcore_barrier()
        for b in range(R // CHUNK):
            off = sid * R + b * CHUNK
            pltpu.sync_copy(zacc.at[pl.ds(off, CHUNK)], bufA)
            pltpu.sync_copy(bufA, z_out.at[cid, pl.ds(off, CHUNK)])
            pltpu.sync_copy(wacc.at[pl.ds(off, CHUNK)], ndv0)
            pltpu.sync_copy(ndv0, w_out.at[cid, pl.ds(off, CHUNK)])

    return k(comb, u, nd)


def _tc_final(zp, wp, stats, b1, W2, b2, Wl, bl):
    def body(zp_ref, wp_ref, st_ref, b1_ref, W2_ref, b2_ref, Wl_ref, bl_ref,
             out_ref):
        z = (zp_ref[0] + zp_ref[1])[0:N, :]
        nd = st_ref[1, 0:N]
        h1 = jnp.maximum(z * nd[:, None] + b1_ref[...][None, :], 0.0)
        wsum = wp_ref[0] + wp_ref[1]
        w = (st_ref[0, 0:N] * wsum[0:N])[None, :]
        q = jnp.dot(w, h1, preferred_element_type=jnp.float32)
        v2 = jnp.dot(W2_ref[...], Wl_ref[...],
                     preferred_element_type=jnp.float32)
        head = jnp.dot(b2_ref[...][None, :], Wl_ref[...],
                       preferred_element_type=jnp.float32)
        out_ref[...] = (jnp.dot(q, v2, preferred_element_type=jnp.float32)
                        * (1.0 / N) + head + bl_ref[...][None, :])

    return pl.pallas_call(
        body,
        out_shape=jax.ShapeDtypeStruct((1, 1), jnp.float32),
    )(zp, wp, stats, b1, W2, b2, Wl, bl)


def kernel(x, edge_index, W1, b1, W2, b2, gamma, beta, Wl, bl):
    E = edge_index.shape[1]
    ei = edge_index.astype(jnp.int32)
    ept = -(-E // (NW * CHUNK)) * CHUNK
    if (ept // CHUNK) % 2:
        ept += CHUNK
    EP = ept * NW
    C = ept // CHUNK
    pad = jnp.full((EP - E,), N, jnp.int32)
    srcp = jnp.concatenate([ei[0], pad]).reshape(NW, C, 1, CHUNK)
    dstp = jnp.concatenate([ei[1], pad]).reshape(NW, C, 1, CHUNK)
    comb = jnp.concatenate([srcp, dstp], axis=2)

    degOp, degIp = _sc_degrees(comb)
    u, stats = _tc_prep(x, W1, gamma, beta, degOp, degIp)
    zp, wp = _sc_aggregate(comb, u, stats[1])
    return _tc_final(zp, wp, stats, b1, W2, b2, Wl, bl)


# speedup vs baseline: 8.0724x; 1.0109x over previous
import functools

import jax
import jax.numpy as jnp
from jax import lax
from jax.experimental import pallas as pl
from jax.experimental.pallas import tpu as pltpu
from jax.experimental.pallas import tpu_sc as plsc

N = 10000
D = 128
H = 128
NP = 10240
NC = 2
NS = 16
L = 16
NW = NC * NS
CHUNK = 128


def _sc_degrees(comb):
    C = comb.shape[1]
    mesh = plsc.VectorSubcoreMesh(core_axis_name="c", subcore_axis_name="s")

    @functools.partial(
        pl.kernel,
        out_type=(jax.ShapeDtypeStruct((NW, NP), jnp.float32),
                  jax.ShapeDtypeStruct((NW, NP), jnp.float32)),
        mesh=mesh,
        compiler_params=pltpu.CompilerParams(needs_layout_passes=False),
        scratch_types=[
            pltpu.VMEM((C, 2, CHUNK), jnp.int32),
            pltpu.VMEM((NP,), jnp.float32),
            pltpu.VMEM((NP,), jnp.float32),
        ],
    )
    def k(comb_hbm, outO, outI, idx_v, dO_v, dI_v):
        cid = lax.axis_index("c")
        sid = lax.axis_index("s")
        wid = sid * NC + cid
        pltpu.sync_copy(comb_hbm.at[wid], idx_v)

        zeros = jnp.zeros((L,), jnp.float32)

        def zbody(i, _):
            dO_v[pl.ds(i * L, L)] = zeros
            dI_v[pl.ds(i * L, L)] = zeros
            return 0

        lax.fori_loop(0, NP // L, zbody, 0)

        ones = jnp.ones((L,), jnp.float32)

        def body(c, _):
            for g in range(CHUNK // L):
                s16 = idx_v[c, 0, pl.ds(g * L, L)]
                plsc.addupdate_scatter(dO_v, [s16], ones)
                d16 = idx_v[c, 1, pl.ds(g * L, L)]
                plsc.addupdate_scatter(dI_v, [d16], ones)
            return 0

        lax.fori_loop(0, C, body, 0)
        pltpu.sync_copy(dO_v, outO.at[wid])
        pltpu.sync_copy(dI_v, outI.at[wid])

    return k(comb)


def _tc_prep(x, W1, gamma, beta, degOp, degIp):
    def body(x_ref, w1_ref, g_ref, b_ref, dO_ref, dI_ref, u_ref, st_ref):
        xv = x_ref[...]
        mean = jnp.mean(xv, axis=0, keepdims=True)
        var = jnp.mean((xv - mean) ** 2, axis=0, keepdims=True)
        h = (xv - mean) * lax.rsqrt(var + 1e-5) * g_ref[...][None, :] \
            + b_ref[...][None, :]
        y = jnp.dot(h, w1_ref[...], preferred_element_type=jnp.float32)

        degO = jnp.sum(dO_ref[...], axis=0)
        degI = jnp.sum(dI_ref[...], axis=0)
        ns = lax.rsqrt(jnp.where(degO > 0, degO, 1.0))
        nd = lax.rsqrt(jnp.where(degI > 0, degI, 1.0))
        st_ref[0, :] = ns
        st_ref[1, :] = nd

        u_ref[0:N, :] = y * ns[0:N][:, None]
        u_ref[N:NP, :] = jnp.zeros((NP - N, H), jnp.float32)

    return pl.pallas_call(
        body,
        out_shape=(jax.ShapeDtypeStruct((NP, H), jnp.float32),
                   jax.ShapeDtypeStruct((2, NP), jnp.float32)),
    )(x, W1, gamma, beta, degOp, degIp)


def _sc_aggregate(comb, u, nd):
    C = comb.shape[1]
    assert C % 4 == 0 and C >= 8
    R = NP // NS
    mesh = plsc.VectorSubcoreMesh(core_axis_name="c", subcore_axis_name="s")

    @functools.partial(
        pl.kernel,
        out_type=(jax.ShapeDtypeStruct((NC, NP, H), jnp.float32),
                  jax.ShapeDtypeStruct((NC, NP), jnp.float32)),
        mesh=mesh,
        compiler_params=pltpu.CompilerParams(needs_layout_passes=False),
        scratch_types=[
            pltpu.VMEM((4, 2, CHUNK), jnp.int32),
            pltpu.VMEM((2, CHUNK), jnp.float32),
            pltpu.VMEM((CHUNK, H), jnp.float32),
            pltpu.VMEM((CHUNK, H), jnp.float32),
            pltpu.VMEM_SHARED((NP, H), jnp.float32),
            pltpu.VMEM_SHARED((NP,), jnp.float32),
            pltpu.SemaphoreType.DMA,
            pltpu.SemaphoreType.DMA,
            pltpu.SemaphoreType.DMA,
            pltpu.SemaphoreType.DMA,
            pltpu.SemaphoreType.DMA,
            pltpu.SemaphoreType.DMA,
            pltpu.SemaphoreType.DMA,
            pltpu.SemaphoreType.DMA,
        ],
    )
    def k(comb_hbm, u_hbm, nd_hbm, z_out, w_out,
          idxr, ndvr, bufA, bufB, zacc, wacc,
          semA, semB, semI0, semI1, semI2, semI3, semN0, semN1):
        cid = lax.axis_index("c")
        sid = lax.axis_index("s")
        wid = sid * NC + cid

        zeros = jnp.zeros((L,), jnp.float32)

        def zb(r, _):
            for g in range(H // L):
                bufA[r, pl.ds(g * L, L)] = zeros
            return 0

        lax.fori_loop(0, CHUNK, zb, 0)

        for b in range(R // CHUNK):
            pltpu.sync_copy(bufA, zacc.at[pl.ds(sid * R + b * CHUNK, CHUNK)])
            pltpu.sync_copy(bufA.at[0],
                            wacc.at[pl.ds(sid * R + b * CHUNK, CHUNK)])

        semI = [semI0, semI1, semI2, semI3]

        def uwait(buf, sem):
            pltpu.make_async_copy(u_hbm.at[pl.ds(0, CHUNK)], buf, sem).wait()

        def ndwait(ndv, sem):
            pltpu.make_async_copy(nd_hbm.at[pl.ds(0, CHUNK)], ndv, sem).wait()

        def idxwait(s):
            pltpu.make_async_copy(comb_hbm.at[0, 0], idxr.at[s], semI[s]).wait()

        pltpu.sync_copy(comb_hbm.at[wid, 0], idxr.at[0])
        pltpu.async_copy(comb_hbm.at[wid, 1], idxr.at[1], semI[1])
        pltpu.async_copy(comb_hbm.at[wid, 2], idxr.at[2], semI[2])
        pltpu.async_copy(comb_hbm.at[wid, 3], idxr.at[3], semI[3])
        pltpu.async_copy(u_hbm.at[idxr.at[0, 0]], bufA, semA)
        pltpu.async_copy(nd_hbm.at[idxr.at[0, 1]], ndvr.at[0], semN0)
        idxwait(1)
        pltpu.async_copy(u_hbm.at[idxr.at[1, 0]], bufB, semB)
        pltpu.async_copy(nd_hbm.at[idxr.at[1, 1]], ndvr.at[1], semN1)
        plsc.subcore_barrier()

        def body(i, _):
            q0 = 4 * i
            for k in range(4):
                q = q0 + k
                buf, semU = (bufA, semA) if k % 2 == 0 else (bufB, semB)
                nslot, semN = (0, semN0) if k % 2 == 0 else (1, semN1)
                islot = k
                i2 = (k + 2) % 4
                uwait(buf, semU)
                pltpu.sync_copy(buf, zacc.at[idxr.at[islot, 1]], add=True)
                ndwait(ndvr.at[nslot], semN)
                pltpu.sync_copy(ndvr.at[nslot], wacc.at[idxr.at[islot, 0]],
                                add=True)

                @pl.when(q + 4 < C)
                def _():
                    pltpu.async_copy(comb_hbm.at[wid, q + 4], idxr.at[islot],
                                     semI[islot])

                @pl.when(q + 2 < C)
                def _():
                    idxwait(i2)
                    pltpu.async_copy(u_hbm.at[idxr.at[i2, 0]], buf, semU)
                    pltpu.async_copy(nd_hbm.at[idxr.at[i2, 1]], ndvr.at[nslot],
                                     semN)

            return 0

        lax.fori_loop(0, C // 4, body, 0)

        plsc.subcore_barrier()
        for b in range(R // CHUNK):
            off = sid * R + b * CHUNK
            pltpu.sync_copy(zacc.at[pl.ds(off, CHUNK)], bufA)
            pltpu.sync_copy(bufA, z_out.at[cid, pl.ds(off, CHUNK)])
            pltpu.sync_copy(wacc.at[pl.ds(off, CHUNK)], ndvr.at[0])
            pltpu.sync_copy(ndvr.at[0], w_out.at[cid, pl.ds(off, CHUNK)])

    return k(comb, u, nd)


def _tc_final(zp, wp, stats, b1, W2, b2, Wl, bl):
    def body(zp_ref, wp_ref, st_ref, b1_ref, W2_ref, b2_ref, Wl_ref, bl_ref,
             out_ref):
        z = (zp_ref[0] + zp_ref[1])[0:N, :]
        nd = st_ref[1, 0:N]
        h1 = jnp.maximum(z * nd[:, None] + b1_ref[...][None, :], 0.0)
        wsum = wp_ref[0] + wp_ref[1]
        w = (st_ref[0, 0:N] * wsum[0:N])[None, :]
        q = jnp.dot(w, h1, preferred_element_type=jnp.float32)
        v2 = jnp.dot(W2_ref[...], Wl_ref[...],
                     preferred_element_type=jnp.float32)
        head = jnp.dot(b2_ref[...][None, :], Wl_ref[...],
                       preferred_element_type=jnp.float32)
        out_ref[...] = (jnp.dot(q, v2, preferred_element_type=jnp.float32)
                        * (1.0 / N) + head + bl_ref[...][None, :])

    return pl.pallas_call(
        body,
        out_shape=jax.ShapeDtypeStruct((1, 1), jnp.float32),
    )(zp, wp, stats, b1, W2, b2, Wl, bl)


def kernel(x, edge_index, W1, b1, W2, b2, gamma, beta, Wl, bl):
    E = edge_index.shape[1]
    ei = edge_index.astype(jnp.int32)
    ept = -(-E // (NW * CHUNK)) * CHUNK
    while (ept // CHUNK) % 4 or ept // CHUNK < 8:
        ept += CHUNK
    EP = ept * NW
    C = ept // CHUNK
    pad = jnp.full((EP - E,), N, jnp.int32)
    srcp = jnp.concatenate([ei[0], pad]).reshape(NW, C, 1, CHUNK)
    dstp = jnp.concatenate([ei[1], pad]).reshape(NW, C, 1, CHUNK)
    comb = jnp.concatenate([srcp, dstp], axis=2)

    degOp, degIp = _sc_degrees(comb)
    u, stats = _tc_prep(x, W1, gamma, beta, degOp, degIp)
    zp, wp = _sc_aggregate(comb, u, stats[1])
    return _tc_final(zp, wp, stats, b1, W2, b2, Wl, bl)


# speedup vs baseline: 9.8336x; 1.2182x over previous
import functools

import jax
import jax.numpy as jnp
from jax import lax
from jax.experimental import pallas as pl
from jax.experimental.pallas import tpu as pltpu
from jax.experimental.pallas import tpu_sc as plsc

N = 10000
D = 128
H = 128
NP = 10240
NC = 2
NS = 16
L = 16
HH = H // NC
CHUNK = 128


def _sc_degrees(comb):
    C2 = comb.shape[1]
    Ch = C2 // NC
    mesh = plsc.VectorSubcoreMesh(core_axis_name="c", subcore_axis_name="s")

    @functools.partial(
        pl.kernel,
        out_type=(jax.ShapeDtypeStruct((NC, NS, NP), jnp.float32),
                  jax.ShapeDtypeStruct((NC, NS, NP), jnp.float32)),
        mesh=mesh,
        compiler_params=pltpu.CompilerParams(needs_layout_passes=False),
        scratch_types=[
            pltpu.VMEM((Ch, 2, CHUNK), jnp.int32),
            pltpu.VMEM((NP,), jnp.float32),
            pltpu.VMEM((NP,), jnp.float32),
        ],
    )
    def k(comb_hbm, outO, outI, idx_v, dO_v, dI_v):
        cid = lax.axis_index("c")
        sid = lax.axis_index("s")
        pltpu.sync_copy(comb_hbm.at[sid, pl.ds(cid * Ch, Ch)], idx_v)

        zeros = jnp.zeros((L,), jnp.float32)

        def zbody(i, _):
            dO_v[pl.ds(i * L, L)] = zeros
            dI_v[pl.ds(i * L, L)] = zeros
            return 0

        lax.fori_loop(0, NP // L, zbody, 0)

        ones = jnp.ones((L,), jnp.float32)

        def body(c, _):
            for g in range(CHUNK // L):
                s16 = idx_v[c, 0, pl.ds(g * L, L)]
                plsc.addupdate_scatter(dO_v, [s16], ones)
                d16 = idx_v[c, 1, pl.ds(g * L, L)]
                plsc.addupdate_scatter(dI_v, [d16], ones)
            return 0

        lax.fori_loop(0, Ch, body, 0)
        pltpu.sync_copy(dO_v, outO.at[cid, sid])
        pltpu.sync_copy(dI_v, outI.at[cid, sid])

    return k(comb)


def _tc_prep(x, W1, gamma, beta, degOp, degIp):
    def body(x_ref, w1_ref, g_ref, b_ref, dO_ref, dI_ref, u_ref, st_ref):
        xv = x_ref[...]
        mean = jnp.mean(xv, axis=0, keepdims=True)
        var = jnp.mean((xv - mean) ** 2, axis=0, keepdims=True)
        h = (xv - mean) * lax.rsqrt(var + 1e-5) * g_ref[...][None, :] \
            + b_ref[...][None, :]
        y = jnp.dot(h, w1_ref[...], preferred_element_type=jnp.float32)

        degO = jnp.sum(dO_ref[...], axis=(0, 1))
        degI = jnp.sum(dI_ref[...], axis=(0, 1))
        ns = lax.rsqrt(jnp.where(degO > 0, degO, 1.0))
        nd = lax.rsqrt(jnp.where(degI > 0, degI, 1.0))
        st_ref[0, :] = ns
        st_ref[1, :] = nd

        u = y * ns[0:N][:, None]
        for c in range(NC):
            u_ref[c, 0:N, :] = u[:, c * HH:(c + 1) * HH]
            u_ref[c, N:NP, :] = jnp.zeros((NP - N, HH), jnp.float32)

    return pl.pallas_call(
        body,
        out_shape=(jax.ShapeDtypeStruct((NC, NP, HH), jnp.float32),
                   jax.ShapeDtypeStruct((2, NP), jnp.float32)),
    )(x, W1, gamma, beta, degOp, degIp)


def _sc_aggregate(comb, u2, nd):
    C2 = comb.shape[1]
    assert C2 % 8 == 0 and C2 >= 16
    R = NP // NS
    mesh = plsc.VectorSubcoreMesh(core_axis_name="c", subcore_axis_name="s")

    @functools.partial(
        pl.kernel,
        out_type=(jax.ShapeDtypeStruct((NC, NP, HH), jnp.float32),
                  jax.ShapeDtypeStruct((NC, NP), jnp.float32)),
        mesh=mesh,
        compiler_params=pltpu.CompilerParams(needs_layout_passes=False,
                                             use_tc_tiling_on_sc=False),
        scratch_types=[
            pltpu.VMEM((8, 2, CHUNK), jnp.int32),
            pltpu.VMEM((8, CHUNK), jnp.float32),
            pltpu.VMEM((8, CHUNK, HH), jnp.float32),
            pltpu.VMEM_SHARED((NP, HH), jnp.float32),
            pltpu.VMEM_SHARED((NP,), jnp.float32),
        ] + [pltpu.SemaphoreType.DMA] * 24,
    )
    def k(comb_hbm, uA_hbm, uB_hbm, nd_hbm, z_out, w_out,
          idxr, ndvr, bufr, zacc, wacc, *sems):
        semU = sems[0:8]
        semI = sems[8:16]
        semN = sems[16:24]
        cid = lax.axis_index("c")
        sid = lax.axis_index("s")

        zeros = jnp.zeros((L,), jnp.float32)

        def zb(r, _):
            for g in range(HH // L):
                bufr[0, r, pl.ds(g * L, L)] = zeros
            return 0

        lax.fori_loop(0, CHUNK, zb, 0)

        for b in range(R // CHUNK):
            sl = pl.ds(sid * R + b * CHUNK, CHUNK)
            pltpu.sync_copy(bufr.at[0], zacc.at[sl])
            pltpu.sync_copy(bufr.at[0, 0], wacc.at[pl.ds(sid * R + b * CHUNK,
                                                         HH)])
            pltpu.sync_copy(bufr.at[0, 0],
                            wacc.at[pl.ds(sid * R + b * CHUNK + HH, HH)])

        def uwait(s):
            pltpu.make_async_copy(uA_hbm.at[pl.ds(0, CHUNK)], bufr.at[s],
                                  semU[s]).wait()

        def ugather(q, s):
            @pl.when(cid == 0)
            def _():
                pltpu.async_copy(uA_hbm.at[idxr.at[s, 0]], bufr.at[s], semU[s])

            @pl.when(cid == 1)
            def _():
                pltpu.async_copy(uB_hbm.at[idxr.at[s, 0]], bufr.at[s], semU[s])

        def ndwait(s):
            pltpu.make_async_copy(nd_hbm.at[pl.ds(0, CHUNK)], ndvr.at[s],
                                  semN[s]).wait()

        def ndgather(q, s):
            pltpu.async_copy(nd_hbm.at[idxr.at[s, 1]], ndvr.at[s], semN[s])

        def idxwait(s):
            pltpu.make_async_copy(comb_hbm.at[0, 0], idxr.at[s], semI[s]).wait()

        pltpu.sync_copy(comb_hbm.at[sid, 0], idxr.at[0])
        for j in range(1, 8):
            pltpu.async_copy(comb_hbm.at[sid, j], idxr.at[j], semI[j])
        for j in range(6):
            if j > 0:
                idxwait(j)
            ugather(j, j)
            ndgather(j, j)
        plsc.subcore_barrier()

        def body(i, _):
            q0 = 8 * i
            for k in range(8):
                q = q0 + k
                s = k
                s6 = (k + 6) % 8
                uwait(s)
                pltpu.sync_copy(bufr.at[s], zacc.at[idxr.at[s, 1]], add=True)
                ndwait(s)
                pltpu.sync_copy(ndvr.at[s], wacc.at[idxr.at[s, 0]], add=True)

                @pl.when(q + 8 < C2)
                def _():
                    pltpu.async_copy(comb_hbm.at[sid, q + 8], idxr.at[s],
                                     semI[s])

                @pl.when(q + 6 < C2)
                def _():
                    idxwait(s6)
                    ugather(q + 6, s6)
                    ndgather(q + 6, s6)

            return 0

        lax.fori_loop(0, C2 // 8, body, 0)

        plsc.subcore_barrier()
        for b in range(R // CHUNK):
            sl = pl.ds(sid * R + b * CHUNK, CHUNK)
            pltpu.sync_copy(zacc.at[sl], bufr.at[0])
            pltpu.sync_copy(bufr.at[0], z_out.at[cid, sl])
            pltpu.sync_copy(wacc.at[sl], ndvr.at[0])
            pltpu.sync_copy(ndvr.at[0], w_out.at[cid, sl])

    return k(comb, u2[0], u2[1], nd)


def _tc_final(zp, wp, stats, b1, W2, b2, Wl, bl):
    def body(zp_ref, wp_ref, st_ref, b1_ref, W2_ref, b2_ref, Wl_ref, bl_ref,
             out_ref):
        z = jnp.concatenate([zp_ref[0], zp_ref[1]], axis=1)[0:N, :]
        nd = st_ref[1, 0:N]
        h1 = jnp.maximum(z * nd[:, None] + b1_ref[...][None, :], 0.0)
        wsum = (wp_ref[0] + wp_ref[1]) * 0.5
        w = (st_ref[0, 0:N] * wsum[0:N])[None, :]
        q = jnp.dot(w, h1, preferred_element_type=jnp.float32)
        v2 = jnp.dot(W2_ref[...], Wl_ref[...],
                     preferred_element_type=jnp.float32)
        head = jnp.dot(b2_ref[...][None, :], Wl_ref[...],
                       preferred_element_type=jnp.float32)
        out_ref[...] = (jnp.dot(q, v2, preferred_element_type=jnp.float32)
                        * (1.0 / N) + head + bl_ref[...][None, :])

    return pl.pallas_call(
        body,
        out_shape=jax.ShapeDtypeStruct((1, 1), jnp.float32),
    )(zp, wp, stats, b1, W2, b2, Wl, bl)


def kernel(x, edge_index, W1, b1, W2, b2, gamma, beta, Wl, bl):
    E = edge_index.shape[1]
    ei = edge_index.astype(jnp.int32)
    ept = -(-E // (NS * CHUNK)) * CHUNK
    while (ept // CHUNK) % 8 or ept // CHUNK < 16:
        ept += CHUNK
    EP = ept * NS
    C2 = ept // CHUNK
    pad = jnp.full((EP - E,), N, jnp.int32)
    srcp = jnp.concatenate([ei[0], pad]).reshape(NS, C2, 1, CHUNK)
    dstp = jnp.concatenate([ei[1], pad]).reshape(NS, C2, 1, CHUNK)
    comb = jnp.concatenate([srcp, dstp], axis=2)

    degOp, degIp = _sc_degrees(comb)
    u2, stats = _tc_prep(x, W1, gamma, beta, degOp, degIp)
    zp, wp = _sc_aggregate(comb, u2, stats[1])
    return _tc_final(zp, wp, stats, b1, W2, b2, Wl, bl)


# speedup vs baseline: 10.6876x; 1.0869x over previous
import functools

import jax
import jax.numpy as jnp
from jax import lax
from jax.experimental import pallas as pl
from jax.experimental.pallas import tpu as pltpu
from jax.experimental.pallas import tpu_sc as plsc

N = 10000
D = 128
H = 128
NP = 10240
NC = 2
NS = 16
L = 16
HH = H // NC
CHUNK = 128


def _sc_degrees(comb):
    C2 = comb.shape[1]
    Ch = C2 // NC
    mesh = plsc.VectorSubcoreMesh(core_axis_name="c", subcore_axis_name="s")

    @functools.partial(
        pl.kernel,
        out_type=(jax.ShapeDtypeStruct((NC, NS, NP), jnp.float32),
                  jax.ShapeDtypeStruct((NC, NS, NP), jnp.float32)),
        mesh=mesh,
        compiler_params=pltpu.CompilerParams(needs_layout_passes=False),
        scratch_types=[
            pltpu.VMEM((Ch, 2, CHUNK), jnp.int32),
            pltpu.VMEM((NP,), jnp.float32),
            pltpu.VMEM((NP,), jnp.float32),
        ],
    )
    def k(comb_hbm, outO, outI, idx_v, dO_v, dI_v):
        cid = lax.axis_index("c")
        sid = lax.axis_index("s")
        pltpu.sync_copy(comb_hbm.at[sid, pl.ds(cid * Ch, Ch)], idx_v)

        zeros = jnp.zeros((L,), jnp.float32)

        def zbody(i, _):
            dO_v[pl.ds(i * L, L)] = zeros
            dI_v[pl.ds(i * L, L)] = zeros
            return 0

        lax.fori_loop(0, NP // L, zbody, 0)

        ones = jnp.ones((L,), jnp.float32)

        def body(c, _):
            for g in range(CHUNK // L):
                s16 = idx_v[c, 0, pl.ds(g * L, L)]
                plsc.addupdate_scatter(dO_v, [s16], ones)
                d16 = idx_v[c, 1, pl.ds(g * L, L)]
                plsc.addupdate_scatter(dI_v, [d16], ones)
            return 0

        lax.fori_loop(0, Ch, body, 0)
        pltpu.sync_copy(dO_v, outO.at[cid, sid])
        pltpu.sync_copy(dI_v, outI.at[cid, sid])

    return k(comb)


def _tc_prep(x, W1, gamma, beta, degOp, degIp):
    def body(x_ref, w1_ref, g_ref, b_ref, dO_ref, dI_ref, u_ref, st_ref):
        xv = x_ref[...]
        mean = jnp.mean(xv, axis=0, keepdims=True)
        var = jnp.mean((xv - mean) ** 2, axis=0, keepdims=True)
        h = (xv - mean) * lax.rsqrt(var + 1e-5) * g_ref[...][None, :] \
            + b_ref[...][None, :]
        y = jnp.dot(h, w1_ref[...], preferred_element_type=jnp.float32)

        degO = jnp.sum(dO_ref[...], axis=(0, 1))
        degI = jnp.sum(dI_ref[...], axis=(0, 1))
        ns = lax.rsqrt(jnp.where(degO > 0, degO, 1.0))
        nd = lax.rsqrt(jnp.where(degI > 0, degI, 1.0))
        st_ref[0, :] = ns
        st_ref[1, :] = nd

        u = y * ns[0:N][:, None]
        for c in range(NC):
            u_ref[c, 0:N, :] = u[:, c * HH:(c + 1) * HH]
            u_ref[c, N:NP, :] = jnp.zeros((NP - N, HH), jnp.float32)

    return pl.pallas_call(
        body,
        out_shape=(jax.ShapeDtypeStruct((NC, NP, HH), jnp.float32),
                   jax.ShapeDtypeStruct((2, NP), jnp.float32)),
    )(x, W1, gamma, beta, degOp, degIp)


def _sc_aggregate(comb, u2, nd):
    C2 = comb.shape[1]
    assert C2 % 8 == 0 and C2 >= 16
    R = NP // NS
    mesh = plsc.VectorSubcoreMesh(core_axis_name="c", subcore_axis_name="s")

    @functools.partial(
        pl.kernel,
        out_type=(jax.ShapeDtypeStruct((NC, NP, HH), jnp.float32),
                  jax.ShapeDtypeStruct((NC, NP), jnp.float32)),
        mesh=mesh,
        compiler_params=pltpu.CompilerParams(needs_layout_passes=False,
                                             use_tc_tiling_on_sc=False),
        scratch_types=[
            pltpu.VMEM((8, 2, CHUNK), jnp.int32),
            pltpu.VMEM((8, CHUNK), jnp.float32),
            pltpu.VMEM((8, CHUNK, HH), jnp.float32),
            pltpu.VMEM_SHARED((NP, HH), jnp.float32),
            pltpu.VMEM_SHARED((NP,), jnp.float32),
        ] + [pltpu.SemaphoreType.DMA] * 24,
    )
    def k(comb_hbm, uA_hbm, uB_hbm, nd_hbm, z_out, w_out,
          idxr, ndvr, bufr, zacc, wacc, *sems):
        semU = sems[0:8]
        semI = sems[8:16]
        semN = sems[16:24]
        cid = lax.axis_index("c")
        sid = lax.axis_index("s")

        zeros = jnp.zeros((L,), jnp.float32)

        def zb(r, _):
            for g in range(HH // L):
                bufr[0, r, pl.ds(g * L, L)] = zeros
            return 0

        lax.fori_loop(0, CHUNK, zb, 0)

        for b in range(R // CHUNK):
            sl = pl.ds(sid * R + b * CHUNK, CHUNK)
            pltpu.sync_copy(bufr.at[0], zacc.at[sl])
            pltpu.sync_copy(bufr.at[0, 0], wacc.at[pl.ds(sid * R + b * CHUNK,
                                                         HH)])
            pltpu.sync_copy(bufr.at[0, 0],
                            wacc.at[pl.ds(sid * R + b * CHUNK + HH, HH)])

        def uwait(s):
            pltpu.make_async_copy(uA_hbm.at[pl.ds(0, CHUNK)], bufr.at[s],
                                  semU[s]).wait()

        def ugather(q, s):
            @pl.when(cid == 0)
            def _():
                pltpu.async_copy(uA_hbm.at[idxr.at[s, 0]], bufr.at[s], semU[s])

            @pl.when(cid == 1)
            def _():
                pltpu.async_copy(uB_hbm.at[idxr.at[s, 0]], bufr.at[s], semU[s])

        def ndwait(s):
            pltpu.make_async_copy(nd_hbm.at[pl.ds(0, CHUNK)], ndvr.at[s],
                                  semN[s]).wait()

        def ndgather(q, s):
            @pl.when(cid == (q % 2))
            def _():
                pltpu.async_copy(nd_hbm.at[idxr.at[s, 1]], ndvr.at[s], semN[s])

        def idxwait(s):
            pltpu.make_async_copy(comb_hbm.at[0, 0], idxr.at[s], semI[s]).wait()

        pltpu.sync_copy(comb_hbm.at[sid, 0], idxr.at[0])
        for j in range(1, 8):
            pltpu.async_copy(comb_hbm.at[sid, j], idxr.at[j], semI[j])
        for j in range(6):
            if j > 0:
                idxwait(j)
            ugather(j, j)
            ndgather(j, j)
        plsc.subcore_barrier()

        def body(i, _):
            q0 = 8 * i
            for k in range(8):
                q = q0 + k
                s = k
                s6 = (k + 6) % 8
                uwait(s)
                pltpu.sync_copy(bufr.at[s], zacc.at[idxr.at[s, 1]], add=True)

                @pl.when(cid == (k % 2))
                def _():
                    ndwait(s)
                    pltpu.sync_copy(ndvr.at[s], wacc.at[idxr.at[s, 0]],
                                    add=True)

                @pl.when(q + 8 < C2)
                def _():
                    pltpu.async_copy(comb_hbm.at[sid, q + 8], idxr.at[s],
                                     semI[s])

                @pl.when(q + 6 < C2)
                def _():
                    idxwait(s6)
                    ugather(q + 6, s6)
                    ndgather(q + 6, s6)

            return 0

        lax.fori_loop(0, C2 // 8, body, 0)

        plsc.subcore_barrier()
        for b in range(R // CHUNK):
            sl = pl.ds(sid * R + b * CHUNK, CHUNK)
            pltpu.sync_copy(zacc.at[sl], bufr.at[0])
            pltpu.sync_copy(bufr.at[0], z_out.at[cid, sl])
            pltpu.sync_copy(wacc.at[sl], ndvr.at[0])
            pltpu.sync_copy(ndvr.at[0], w_out.at[cid, sl])

    return k(comb, u2[0], u2[1], nd)


def _tc_final(zp, wp, stats, b1, W2, b2, Wl, bl):
    def body(zp_ref, wp_ref, st_ref, b1_ref, W2_ref, b2_ref, Wl_ref, bl_ref,
             out_ref):
        z = jnp.concatenate([zp_ref[0], zp_ref[1]], axis=1)[0:N, :]
        nd = st_ref[1, 0:N]
        h1 = jnp.maximum(z * nd[:, None] + b1_ref[...][None, :], 0.0)
        wsum = wp_ref[0] + wp_ref[1]
        w = (st_ref[0, 0:N] * wsum[0:N])[None, :]
        q = jnp.dot(w, h1, preferred_element_type=jnp.float32)
        v2 = jnp.dot(W2_ref[...], Wl_ref[...],
                     preferred_element_type=jnp.float32)
        head = jnp.dot(b2_ref[...][None, :], Wl_ref[...],
                       preferred_element_type=jnp.float32)
        out_ref[...] = (jnp.dot(q, v2, preferred_element_type=jnp.float32)
                        * (1.0 / N) + head + bl_ref[...][None, :])

    return pl.pallas_call(
        body,
        out_shape=jax.ShapeDtypeStruct((1, 1), jnp.float32),
    )(zp, wp, stats, b1, W2, b2, Wl, bl)


def kernel(x, edge_index, W1, b1, W2, b2, gamma, beta, Wl, bl):
    E = edge_index.shape[1]
    ei = edge_index.astype(jnp.int32)
    ept = -(-E // (NS * CHUNK)) * CHUNK
    while (ept // CHUNK) % 8 or ept // CHUNK < 16:
        ept += CHUNK
    EP = ept * NS
    C2 = ept // CHUNK
    pad = jnp.full((EP - E,), N, jnp.int32)
    srcp = jnp.concatenate([ei[0], pad]).reshape(NS, C2, 1, CHUNK)
    dstp = jnp.concatenate([ei[1], pad]).reshape(NS, C2, 1, CHUNK)
    comb = jnp.concatenate([srcp, dstp], axis=2)

    degOp, degIp = _sc_degrees(comb)
    u2, stats = _tc_prep(x, W1, gamma, beta, degOp, degIp)
    zp, wp = _sc_aggregate(comb, u2, stats[1])
    return _tc_final(zp, wp, stats, b1, W2, b2, Wl, bl)


# speedup vs baseline: 11.3443x; 1.0614x over previous
import functools

import jax
import jax.numpy as jnp
from jax import lax
from jax.experimental import pallas as pl
from jax.experimental.pallas import tpu as pltpu
from jax.experimental.pallas import tpu_sc as plsc

N = 10000
D = 128
H = 128
NP = 10240
NC = 2
NS = 16
L = 16
HH = H // NC
CHUNK = 128


def _sc_degrees(comb):
    C2 = comb.shape[1]
    Ch = C2 // NC
    mesh = plsc.VectorSubcoreMesh(core_axis_name="c", subcore_axis_name="s")

    @functools.partial(
        pl.kernel,
        out_type=(jax.ShapeDtypeStruct((NC, NS, NP), jnp.float32),
                  jax.ShapeDtypeStruct((NC, NS, NP), jnp.float32)),
        mesh=mesh,
        compiler_params=pltpu.CompilerParams(needs_layout_passes=False),
        scratch_types=[
            pltpu.VMEM((Ch, 2, CHUNK), jnp.int32),
            pltpu.VMEM((NP,), jnp.float32),
            pltpu.VMEM((NP,), jnp.float32),
        ],
    )
    def k(comb_hbm, outO, outI, idx_v, dO_v, dI_v):
        cid = lax.axis_index("c")
        sid = lax.axis_index("s")
        pltpu.sync_copy(comb_hbm.at[sid, pl.ds(cid * Ch, Ch)], idx_v)

        zeros = jnp.zeros((L,), jnp.float32)

        def zbody(i, _):
            dO_v[pl.ds(i * L, L)] = zeros
            dI_v[pl.ds(i * L, L)] = zeros
            return 0

        lax.fori_loop(0, NP // L, zbody, 0)

        ones = jnp.ones((L,), jnp.float32)

        def body(c, _):
            for g in range(CHUNK // L):
                s16 = idx_v[c, 0, pl.ds(g * L, L)]
                plsc.addupdate_scatter(dO_v, [s16], ones)
                d16 = idx_v[c, 1, pl.ds(g * L, L)]
                plsc.addupdate_scatter(dI_v, [d16], ones)
            return 0

        lax.fori_loop(0, Ch, body, 0)
        pltpu.sync_copy(dO_v, outO.at[cid, sid])
        pltpu.sync_copy(dI_v, outI.at[cid, sid])

    return k(comb)


def _tc_y(x, W1, gamma, beta):
    def body(x_ref, w1_ref, g_ref, b_ref, y_ref):
        xv = x_ref[...]
        mean = jnp.mean(xv, axis=0, keepdims=True)
        var = jnp.mean((xv - mean) ** 2, axis=0, keepdims=True)
        h = (xv - mean) * lax.rsqrt(var + 1e-5) * g_ref[...][None, :] \
            + b_ref[...][None, :]
        y_ref[...] = jnp.dot(h, w1_ref[...], preferred_element_type=jnp.float32)

    return pl.pallas_call(
        body,
        out_shape=jax.ShapeDtypeStruct((N, H), jnp.float32),
    )(x, W1, gamma, beta)


def _tc_norm(y, degOp, degIp):
    def body(y_ref, dO_ref, dI_ref, u_ref, st_ref):
        degO = jnp.sum(dO_ref[...], axis=(0, 1))
        degI = jnp.sum(dI_ref[...], axis=(0, 1))
        ns = lax.rsqrt(jnp.where(degO > 0, degO, 1.0))
        nd = lax.rsqrt(jnp.where(degI > 0, degI, 1.0))
        st_ref[0, :] = ns
        st_ref[1, :] = nd

        u = y_ref[...] * ns[0:N][:, None]
        for c in range(NC):
            u_ref[c, 0:N, :] = u[:, c * HH:(c + 1) * HH]
            u_ref[c, N:NP, :] = jnp.zeros((NP - N, HH), jnp.float32)

    return pl.pallas_call(
        body,
        out_shape=(jax.ShapeDtypeStruct((NC, NP, HH), jnp.float32),
                   jax.ShapeDtypeStruct((2, NP), jnp.float32)),
    )(y, degOp, degIp)


def _sc_aggregate(comb, u2, nd):
    C2 = comb.shape[1]
    assert C2 % 8 == 0 and C2 >= 16
    R = NP // NS
    mesh = plsc.VectorSubcoreMesh(core_axis_name="c", subcore_axis_name="s")

    @functools.partial(
        pl.kernel,
        out_type=(jax.ShapeDtypeStruct((NC, NP, HH), jnp.float32),
                  jax.ShapeDtypeStruct((NC, NP), jnp.float32)),
        mesh=mesh,
        compiler_params=pltpu.CompilerParams(needs_layout_passes=False,
                                             use_tc_tiling_on_sc=False),
        scratch_types=[
            pltpu.VMEM((8, 2, CHUNK), jnp.int32),
            pltpu.VMEM((8, CHUNK), jnp.float32),
            pltpu.VMEM((8, CHUNK, HH), jnp.float32),
            pltpu.VMEM_SHARED((NP, HH), jnp.float32),
            pltpu.VMEM_SHARED((NP,), jnp.float32),
        ] + [pltpu.SemaphoreType.DMA] * 24,
    )
    def k(comb_hbm, uA_hbm, uB_hbm, nd_hbm, z_out, w_out,
          idxr, ndvr, bufr, zacc, wacc, *sems):
        semU = sems[0:8]
        semI = sems[8:16]
        semN = sems[16:24]
        cid = lax.axis_index("c")
        sid = lax.axis_index("s")

        zeros = jnp.zeros((L,), jnp.float32)

        def zb(r, _):
            for g in range(HH // L):
                bufr[0, r, pl.ds(g * L, L)] = zeros
            return 0

        lax.fori_loop(0, CHUNK, zb, 0)

        for b in range(R // CHUNK):
            sl = pl.ds(sid * R + b * CHUNK, CHUNK)
            pltpu.sync_copy(bufr.at[0], zacc.at[sl])
            pltpu.sync_copy(bufr.at[0, 0], wacc.at[pl.ds(sid * R + b * CHUNK,
                                                         HH)])
            pltpu.sync_copy(bufr.at[0, 0],
                            wacc.at[pl.ds(sid * R + b * CHUNK + HH, HH)])

        def uwait(s):
            pltpu.make_async_copy(uA_hbm.at[pl.ds(0, CHUNK)], bufr.at[s],
                                  semU[s]).wait()

        def ugather(q, s):
            @pl.when(cid == 0)
            def _():
                pltpu.async_copy(uA_hbm.at[idxr.at[s, 0]], bufr.at[s], semU[s])

            @pl.when(cid == 1)
            def _():
                pltpu.async_copy(uB_hbm.at[idxr.at[s, 0]], bufr.at[s], semU[s])

        def ndwait(s):
            pltpu.make_async_copy(nd_hbm.at[pl.ds(0, CHUNK)], ndvr.at[s],
                                  semN[s]).wait()

        def ndgather(q, s):
            @pl.when(cid == (q % 2))
            def _():
                pltpu.async_copy(nd_hbm.at[idxr.at[s, 1]], ndvr.at[s], semN[s])

        def idxwait(s):
            pltpu.make_async_copy(comb_hbm.at[0, 0], idxr.at[s], semI[s]).wait()

        pltpu.sync_copy(comb_hbm.at[sid, 0], idxr.at[0])
        for j in range(1, 8):
            pltpu.async_copy(comb_hbm.at[sid, j], idxr.at[j], semI[j])
        for j in range(6):
            if j > 0:
                idxwait(j)
            ugather(j, j)
            ndgather(j, j)
        plsc.subcore_barrier()

        def body(i, _):
            q0 = 8 * i
            for k in range(8):
                q = q0 + k
                s = k
                s6 = (k + 6) % 8
                uwait(s)
                pltpu.sync_copy(bufr.at[s], zacc.at[idxr.at[s, 1]], add=True)

                @pl.when(cid == (k % 2))
                def _():
                    ndwait(s)
                    pltpu.sync_copy(ndvr.at[s], wacc.at[idxr.at[s, 0]],
                                    add=True)

                @pl.when(q + 8 < C2)
                def _():
                    pltpu.async_copy(comb_hbm.at[sid, q + 8], idxr.at[s],
                                     semI[s])

                @pl.when(q + 6 < C2)
                def _():
                    idxwait(s6)
                    ugather(q + 6, s6)
                    ndgather(q + 6, s6)

            return 0

        lax.fori_loop(0, C2 // 8, body, 0)

        plsc.subcore_barrier()
        for b in range(R // CHUNK):
            sl = pl.ds(sid * R + b * CHUNK, CHUNK)
            pltpu.sync_copy(zacc.at[sl], bufr.at[0])
            pltpu.sync_copy(bufr.at[0], z_out.at[cid, sl])
            pltpu.sync_copy(wacc.at[sl], ndvr.at[0])
            pltpu.sync_copy(ndvr.at[0], w_out.at[cid, sl])

    return k(comb, u2[0], u2[1], nd)


def _tc_final(zp, wp, stats, b1, W2, b2, Wl, bl):
    def body(zp_ref, wp_ref, st_ref, b1_ref, W2_ref, b2_ref, Wl_ref, bl_ref,
             out_ref):
        z = jnp.concatenate([zp_ref[0], zp_ref[1]], axis=1)[0:N, :]
        nd = st_ref[1, 0:N]
        h1 = jnp.maximum(z * nd[:, None] + b1_ref[...][None, :], 0.0)
        wsum = wp_ref[0] + wp_ref[1]
        w = (st_ref[0, 0:N] * wsum[0:N])[None, :]
        q = jnp.dot(w, h1, preferred_element_type=jnp.float32)
        v2 = jnp.dot(W2_ref[...], Wl_ref[...],
                     preferred_element_type=jnp.float32)
        head = jnp.dot(b2_ref[...][None, :], Wl_ref[...],
                       preferred_element_type=jnp.float32)
        out_ref[...] = (jnp.dot(q, v2, preferred_element_type=jnp.float32)
                        * (1.0 / N) + head + bl_ref[...][None, :])

    return pl.pallas_call(
        body,
        out_shape=jax.ShapeDtypeStruct((1, 1), jnp.float32),
    )(zp, wp, stats, b1, W2, b2, Wl, bl)


def kernel(x, edge_index, W1, b1, W2, b2, gamma, beta, Wl, bl):
    E = edge_index.shape[1]
    ei = edge_index.astype(jnp.int32)
    ept = -(-E // (NS * CHUNK)) * CHUNK
    while (ept // CHUNK) % 8 or ept // CHUNK < 16:
        ept += CHUNK
    EP = ept * NS
    C2 = ept // CHUNK
    pad = jnp.full((EP - E,), N, jnp.int32)
    srcp = jnp.concatenate([ei[0], pad]).reshape(NS, C2, 1, CHUNK)
    dstp = jnp.concatenate([ei[1], pad]).reshape(NS, C2, 1, CHUNK)
    comb = jnp.concatenate([srcp, dstp], axis=2)

    degOp, degIp = _sc_degrees(comb)
    y = _tc_y(x, W1, gamma, beta)
    u2, stats = _tc_norm(y, degOp, degIp)
    zp, wp = _sc_aggregate(comb, u2, stats[1])
    return _tc_final(zp, wp, stats, b1, W2, b2, Wl, bl)
